# Initial kernel scaffold; baseline (speedup 1.0000x reference)
import functools

import jax
import jax.numpy as jnp
from jax import lax
from jax.experimental import pallas as pl
from jax.experimental.pallas import tpu as pltpu
from jax.experimental.pallas import tpu_sc as plsc

N = 10000
D = 128
M = 3
E = 320000

NP = 10240
NPA = 10368
HS = 10496
NW = 32
EPT = 10240
NCH = EPT // 128
BK = 1280
NB = NP // BK


def _mesh():
    return plsc.VectorSubcoreMesh(core_axis_name="c", subcore_axis_name="s")


def _deg_body(dstp, zeros1, ones, degp, dbuf, obuf, hist, sem):
    c = lax.axis_index("c")
    s = lax.axis_index("s")
    wid = s * 2 + c
    pltpu.async_copy(ones, obuf, sem).wait()
    for m in range(M):
        pltpu.async_copy(zeros1, hist.at[pl.ds(s * 656, 656)], sem).wait()
        pltpu.async_copy(dstp.at[m, wid], dbuf, sem).wait()
        plsc.subcore_barrier()

        @pl.loop(0, NCH)
        def _chunk(j):
            pltpu.sync_copy(obuf, hist.at[dbuf.at[j]], add=True)

        plsc.subcore_barrier()
        pltpu.async_copy(hist.at[pl.ds(s * 640, 640)],
                         degp.at[c, m, pl.ds(s * 640, 640)], sem).wait()
        plsc.subcore_barrier()


def _deg_call(dstp, zeros1, ones):
    return pl.kernel(
        _deg_body,
        out_type=jax.ShapeDtypeStruct((2, M, NP), jnp.float32),
        mesh=_mesh(),
        scratch_types=[
            pltpu.VMEM((NCH, 128), jnp.int32),
            pltpu.VMEM((128,), jnp.float32),
            pltpu.VMEM_SHARED((HS,), jnp.float32),
            pltpu.SemaphoreType.DMA,
        ],
    )(dstp, zeros1, ones)


def _agg_body(hpflat, srcp, dstp, zeros2, parts, sbuf, dbuf, rows, acc, sem):
    c = lax.axis_index("c")
    s = lax.axis_index("s")
    wid = s * 2 + c
    for m in range(M):
        pltpu.async_copy(zeros2, acc.at[pl.ds(s * 648, 648)], sem).wait()
        pltpu.async_copy(srcp.at[m, wid], sbuf, sem).wait()
        pltpu.async_copy(dstp.at[m, wid], dbuf, sem).wait()
        plsc.subcore_barrier()

        @pl.loop(0, NCH)
        def _chunk(j):
            pltpu.async_copy(hpflat.at[sbuf.at[j]], rows, sem).wait()
            pltpu.sync_copy(rows, acc.at[dbuf.at[j]], add=True)

        plsc.subcore_barrier()
        pltpu.async_copy(acc.at[pl.ds(s * 640, 640)],
                         parts.at[c, m, pl.ds(s * 640, 640)], sem).wait()
        plsc.subcore_barrier()


def _agg_call(hpflat, srcp, dstp, zeros2):
    return pl.kernel(
        _agg_body,
        out_type=jax.ShapeDtypeStruct((2, M, NP, D), jnp.float32),
        mesh=_mesh(),
        scratch_types=[
            pltpu.VMEM((NCH, 128), jnp.int32),
            pltpu.VMEM((NCH, 128), jnp.int32),
            pltpu.VMEM((128, D), jnp.float32),
            pltpu.VMEM_SHARED((NPA, D), jnp.float32),
            pltpu.SemaphoreType.DMA,
        ],
    )(hpflat, srcp, dstp, zeros2)


def _hp_body(x_ref, w_ref, degp_ref, hp_ref):
    deg = degp_ref[0, 0] + degp_ref[1, 0] + 1.0
    dinv = lax.rsqrt(deg)
    h = jnp.dot(x_ref[...], w_ref[0], preferred_element_type=jnp.float32)
    hp_ref[0] = h * dinv[:, None]


def _hp_call(x_pad, W, degp):
    return pl.pallas_call(
        _hp_body,
        grid=(M, NB),
        in_specs=[
            pl.BlockSpec((BK, D), lambda m, i: (i, 0)),
            pl.BlockSpec((1, D, D), lambda m, i: (m, 0, 0)),
            pl.BlockSpec((2, 1, BK), lambda m, i: (0, m, i)),
        ],
        out_specs=pl.BlockSpec((1, BK, D), lambda m, i: (m, i, 0)),
        out_shape=jax.ShapeDtypeStruct((M, NP, D), jnp.float32),
    )(x_pad, W, degp)


def _emb_body(parts_ref, hp_ref, degp_ref, b_ref, a_ref, wa_ref, ba_ref,
              e_ref, ssum_ref):
    i = pl.program_id(1)
    deg = degp_ref[0, 0] + degp_ref[1, 0] + 1.0
    dinv = lax.rsqrt(deg)
    o = (parts_ref[0, 0] + parts_ref[1, 0] + hp_ref[0]) * dinv[:, None] \
        + b_ref[0][None, :]
    ee = jnp.maximum(o, 0.0) + a_ref[0, 0] * jnp.minimum(o, 0.0)
    e_ref[0] = ee
    t = jnp.tanh(jnp.dot(ee, wa_ref[...], preferred_element_type=jnp.float32)
                 + ba_ref[0][None, :])
    rid = lax.broadcasted_iota(jnp.int32, (BK, D), 0) + i * BK
    t = jnp.where(rid < N, t, 0.0)
    ts = jnp.sum(t, axis=0)

    @pl.when(i == 0)
    def _():
        ssum_ref[0] = ts

    @pl.when(i != 0)
    def _():
        ssum_ref[0] = ssum_ref[0] + ts


def _emb_call(parts, hp, degp, b, prelu_a, Wa, ba):
    return pl.pallas_call(
        _emb_body,
        grid=(M, NB),
        in_specs=[
            pl.BlockSpec((2, 1, BK, D), lambda m, i: (0, m, i, 0)),
            pl.BlockSpec((1, BK, D), lambda m, i: (m, i, 0)),
            pl.BlockSpec((2, 1, BK), lambda m, i: (0, m, i)),
            pl.BlockSpec((1, D), lambda m, i: (m, 0)),
            pl.BlockSpec((1, 1), lambda m, i: (m, 0)),
            pl.BlockSpec((D, D), lambda m, i: (0, 0)),
            pl.BlockSpec((1, D), lambda m, i: (0, 0)),
        ],
        out_specs=[
            pl.BlockSpec((1, BK, D), lambda m, i: (m, i, 0)),
            pl.BlockSpec((1, D), lambda m, i: (m, 0)),
        ],
        out_shape=[
            jax.ShapeDtypeStruct((M, NP, D), jnp.float32),
            jax.ShapeDtypeStruct((M, D), jnp.float32),
        ],
    )(parts, hp, degp, b, prelu_a.reshape(M, 1), Wa, ba.reshape(1, D))


def _comb_body(e_ref, ssum_ref, att_ref, z_ref):
    logits = jnp.sum(ssum_ref[...] * att_ref[...], axis=1) / float(N)
    mx = jnp.max(logits)
    w = jnp.exp(logits - mx)
    beta = w / jnp.sum(w)
    z_ref[...] = (beta[0] * e_ref[0] + beta[1] * e_ref[1]
                  + beta[2] * e_ref[2])


def _comb_call(e, ssum, att):
    return pl.pallas_call(
        _comb_body,
        grid=(NB,),
        in_specs=[
            pl.BlockSpec((M, BK, D), lambda i: (0, i, 0)),
            pl.BlockSpec((M, D), lambda i: (0, 0)),
            pl.BlockSpec((1, D), lambda i: (0, 0)),
        ],
        out_specs=pl.BlockSpec((BK, D), lambda i: (i, 0)),
        out_shape=jax.ShapeDtypeStruct((NP, D), jnp.float32),
    )(e, ssum, att.reshape(1, D))


@jax.jit
def kernel(x, mp_edge_index, W, b, prelu_a, Wa, ba, att):
    src = mp_edge_index[:, 0, :].astype(jnp.int32)
    dst = mp_edge_index[:, 1, :].astype(jnp.int32)
    ept_real = E // NW
    src = src.reshape(M, NW, ept_real)
    dst = dst.reshape(M, NW, ept_real)
    pad = ((0, 0), (0, 0), (0, EPT - ept_real))
    srcp = jnp.pad(src, pad, constant_values=0)
    srcp = srcp + (jnp.arange(M, dtype=jnp.int32) * NP)[:, None, None]
    srcp = srcp.reshape(M, NW, NCH, 128)
    dstp = jnp.pad(dst, pad, constant_values=NP).reshape(M, NW, NCH, 128)

    zeros1 = jnp.zeros((656,), jnp.float32)
    zeros2 = jnp.zeros((648, D), jnp.float32)
    ones = jnp.ones((128,), jnp.float32)
    x_pad = jnp.pad(x, ((0, NP - N), (0, 0)))

    degp = _deg_call(dstp, zeros1, ones)
    hp = _hp_call(x_pad, W, degp)
    parts = _agg_call(hp.reshape(M * NP, D), srcp, dstp, zeros2)
    e, ssum = _emb_call(parts, hp, degp, b, prelu_a, Wa, ba)
    z = _comb_call(e, ssum, att)
    return z[:N]


# speedup vs baseline: 12.9045x; 12.9045x over previous
import functools

import jax
import jax.numpy as jnp
from jax import lax
from jax.experimental import pallas as pl
from jax.experimental.pallas import tpu as pltpu
from jax.experimental.pallas import tpu_sc as plsc

N = 10000
D = 128
M = 3
E = 320000

NP = 10240
NPA = 10368
HS = 10496
NW = 32
EPT = 10240
NCH = EPT // 128
BK = 1280
NB = NP // BK


def _mesh():
    return plsc.VectorSubcoreMesh(core_axis_name="c", subcore_axis_name="s")


def _deg_body(dstp, zeros1, ones, degp, dbuf, obuf, zbuf, hbuf, hist, sem):
    c = lax.axis_index("c")
    s = lax.axis_index("s")
    wid = s * 2 + c
    pltpu.async_copy(ones, obuf, sem).wait()
    pltpu.async_copy(zeros1, zbuf, sem).wait()
    for m in range(M):
        pltpu.async_copy(zbuf, hist.at[pl.ds(s * 656, 656)], sem).wait()
        pltpu.async_copy(dstp.at[m, wid], dbuf, sem).wait()
        plsc.subcore_barrier()

        @pl.loop(0, NCH)
        def _chunk(j):
            pltpu.sync_copy(obuf, hist.at[dbuf.at[j]], add=True)

        plsc.subcore_barrier()
        pltpu.async_copy(hist.at[pl.ds(s * 640, 640)], hbuf, sem).wait()
        pltpu.async_copy(hbuf,
                         degp.at[pl.ds((c * M + m) * NP + s * 640, 640)],
                         sem).wait()
        plsc.subcore_barrier()


def _deg_call(dstp, zeros1, ones):
    return pl.kernel(
        _deg_body,
        out_type=jax.ShapeDtypeStruct((2 * M * NP,), jnp.float32),
        mesh=_mesh(),
        scratch_types=[
            pltpu.VMEM((NCH, 128), jnp.int32),
            pltpu.VMEM((128,), jnp.float32),
            pltpu.VMEM((656,), jnp.float32),
            pltpu.VMEM((640,), jnp.float32),
            pltpu.VMEM_SHARED((HS,), jnp.float32),
            pltpu.SemaphoreType.DMA,
        ],
    )(dstp, zeros1, ones)


def _agg_body(hpflat, srcp, dstp, zeros2, parts, sbuf, dbuf, rows, zbuf,
              acc, sem):
    c = lax.axis_index("c")
    s = lax.axis_index("s")
    wid = s * 2 + c
    pltpu.async_copy(zeros2, zbuf, sem).wait()
    for m in range(M):
        for k in range(8):
            pltpu.async_copy(zbuf, acc.at[pl.ds(s * 648 + k * 81, 81)],
                             sem).wait()
        pltpu.async_copy(srcp.at[m, wid], sbuf, sem).wait()
        pltpu.async_copy(dstp.at[m, wid], dbuf, sem).wait()
        plsc.subcore_barrier()

        @pl.loop(0, NCH)
        def _chunk(j):
            pltpu.async_copy(hpflat.at[sbuf.at[j]], rows, sem).wait()
            pltpu.sync_copy(rows, acc.at[dbuf.at[j]], add=True)

        plsc.subcore_barrier()
        for k in range(5):
            pltpu.async_copy(acc.at[pl.ds(s * 640 + k * 128, 128)], rows,
                             sem).wait()
            pltpu.async_copy(rows,
                             parts.at[c, m, pl.ds(s * 640 + k * 128, 128)],
                             sem).wait()
        plsc.subcore_barrier()


def _agg_call(hpflat, srcp, dstp, zeros2):
    return pl.kernel(
        _agg_body,
        out_type=jax.ShapeDtypeStruct((2, M, NP, D), jnp.float32),
        mesh=_mesh(),
        scratch_types=[
            pltpu.VMEM((NCH, 128), jnp.int32),
            pltpu.VMEM((NCH, 128), jnp.int32),
            pltpu.VMEM((128, D), jnp.float32),
            pltpu.VMEM((81, D), jnp.float32),
            pltpu.VMEM_SHARED((NPA, D), jnp.float32),
            pltpu.SemaphoreType.DMA,
        ],
    )(hpflat, srcp, dstp, zeros2)


def _hp_body(x_ref, w_ref, degp_ref, hp_ref):
    deg = degp_ref[0, :, 0] + degp_ref[0, :, 1] + 1.0
    dinv = lax.rsqrt(deg)
    h = jnp.dot(x_ref[...], w_ref[0], preferred_element_type=jnp.float32)
    hp_ref[0] = h * dinv[:, None]


def _hp_call(x_pad, W, degp_t):
    return pl.pallas_call(
        _hp_body,
        grid=(M, NB),
        in_specs=[
            pl.BlockSpec((BK, D), lambda m, i: (i, 0)),
            pl.BlockSpec((1, D, D), lambda m, i: (m, 0, 0)),
            pl.BlockSpec((1, BK, 2), lambda m, i: (m, i, 0)),
        ],
        out_specs=pl.BlockSpec((1, BK, D), lambda m, i: (m, i, 0)),
        out_shape=jax.ShapeDtypeStruct((M, NP, D), jnp.float32),
    )(x_pad, W, degp_t)


def _emb_body(parts_ref, hp_ref, degp_ref, b_ref, a_ref, wa_ref, ba_ref,
              e_ref, ssum_ref):
    i = pl.program_id(1)
    deg = degp_ref[0, :, 0] + degp_ref[0, :, 1] + 1.0
    dinv = lax.rsqrt(deg)
    o = (parts_ref[0, 0] + parts_ref[1, 0] + hp_ref[0]) * dinv[:, None] \
        + b_ref[0, 0][None, :]
    ee = jnp.maximum(o, 0.0) + a_ref[0, 0, 0] * jnp.minimum(o, 0.0)
    e_ref[0] = ee
    t = jnp.tanh(jnp.dot(ee, wa_ref[...], preferred_element_type=jnp.float32)
                 + ba_ref[0][None, :])
    rid = lax.broadcasted_iota(jnp.int32, (BK, D), 0) + i * BK
    t = jnp.where(rid < N, t, 0.0)
    ts = jnp.sum(t, axis=0)

    @pl.when(i == 0)
    def _():
        ssum_ref[0, 0] = ts

    @pl.when(i != 0)
    def _():
        ssum_ref[0, 0] = ssum_ref[0, 0] + ts


def _emb_call(parts, hp, degp_t, b, prelu_a, Wa, ba):
    return pl.pallas_call(
        _emb_body,
        grid=(M, NB),
        in_specs=[
            pl.BlockSpec((2, 1, BK, D), lambda m, i: (0, m, i, 0)),
            pl.BlockSpec((1, BK, D), lambda m, i: (m, i, 0)),
            pl.BlockSpec((1, BK, 2), lambda m, i: (m, i, 0)),
            pl.BlockSpec((1, 1, D), lambda m, i: (m, 0, 0)),
            pl.BlockSpec((1, 1, 1), lambda m, i: (m, 0, 0)),
            pl.BlockSpec((D, D), lambda m, i: (0, 0)),
            pl.BlockSpec((1, D), lambda m, i: (0, 0)),
        ],
        out_specs=[
            pl.BlockSpec((1, BK, D), lambda m, i: (m, i, 0)),
            pl.BlockSpec((1, 1, D), lambda m, i: (m, 0, 0)),
        ],
        out_shape=[
            jax.ShapeDtypeStruct((M, NP, D), jnp.float32),
            jax.ShapeDtypeStruct((M, 1, D), jnp.float32),
        ],
    )(parts, hp, degp_t, b.reshape(M, 1, D), prelu_a.reshape(M, 1, 1),
      Wa, ba.reshape(1, D))


def _comb_body(e_ref, ssum_ref, att_ref, z_ref):
    logits = jnp.sum(ssum_ref[:, 0, :] * att_ref[...], axis=1) / float(N)
    mx = jnp.max(logits)
    w = jnp.exp(logits - mx)
    beta = w / jnp.sum(w)
    z_ref[...] = (beta[0] * e_ref[0] + beta[1] * e_ref[1]
                  + beta[2] * e_ref[2])


def _comb_call(e, ssum, att):
    return pl.pallas_call(
        _comb_body,
        grid=(NB,),
        in_specs=[
            pl.BlockSpec((M, BK, D), lambda i: (0, i, 0)),
            pl.BlockSpec((M, 1, D), lambda i: (0, 0, 0)),
            pl.BlockSpec((1, D), lambda i: (0, 0)),
        ],
        out_specs=pl.BlockSpec((BK, D), lambda i: (i, 0)),
        out_shape=jax.ShapeDtypeStruct((NP, D), jnp.float32),
    )(e, ssum, att.reshape(1, D))


@jax.jit
def kernel(x, mp_edge_index, W, b, prelu_a, Wa, ba, att):
    src = mp_edge_index[:, 0, :].astype(jnp.int32)
    dst = mp_edge_index[:, 1, :].astype(jnp.int32)
    ept_real = E // NW
    src = src.reshape(M, NW, ept_real)
    dst = dst.reshape(M, NW, ept_real)
    pad = ((0, 0), (0, 0), (0, EPT - ept_real))
    srcp = jnp.pad(src, pad, constant_values=0)
    srcp = srcp + (jnp.arange(M, dtype=jnp.int32) * NP)[:, None, None]
    srcp = srcp.reshape(M, NW, NCH, 128)
    dstp = jnp.pad(dst, pad, constant_values=NP).reshape(M, NW, NCH, 128)

    zeros1 = jnp.zeros((656,), jnp.float32)
    zeros2 = jnp.zeros((81, D), jnp.float32)
    ones = jnp.ones((128,), jnp.float32)
    x_pad = jnp.pad(x, ((0, NP - N), (0, 0)))

    degp = _deg_call(dstp, zeros1, ones).reshape(2, M, NP)
    degp_t = jnp.transpose(degp, (1, 2, 0))
    hp = _hp_call(x_pad, W, degp_t)
    parts = _agg_call(hp.reshape(M * NP, D), srcp, dstp, zeros2)
    e, ssum = _emb_call(parts, hp, degp_t, b, prelu_a, Wa, ba)
    z = _comb_call(e, ssum, att)
    return z[:N]
